# Initial kernel scaffold; baseline (speedup 1.0000x reference)
import jax
import jax.numpy as jnp
from jax.experimental import pallas as pl


def kernel(state, w1, b1, w2, b2, w3, b3):
    raise NotImplementedError("write your pallas kernel here")


# speedup vs baseline: 1.1533x; 1.1533x over previous
import functools

import jax
import jax.numpy as jnp
from jax.experimental import pallas as pl
from jax.experimental.pallas import tpu as pltpu

_PACK = 8
_TM = 4096


def _mlp_kernel(x_ref, w1_ref, b1_ref, w2_ref, b2_ref, w3_ref, b3_ref,
                out_ref):
    x = jnp.dot(x_ref[...], w1_ref[...], preferred_element_type=jnp.float32)
    x = jnp.maximum(x + b1_ref[...], 0.0)
    x = jnp.dot(x, w2_ref[...], preferred_element_type=jnp.float32)
    x = jnp.maximum(x + b2_ref[...], 0.0)
    x = jnp.dot(x, w3_ref[...], preferred_element_type=jnp.float32)
    out_ref[...] = jnp.tanh(x + b3_ref[...]).astype(out_ref.dtype)


def _round_up(x, m):
    return ((x + m - 1) // m) * m


@jax.jit
def _actor_forward(state, w1, b1, w2, b2, w3, b3):
    batch, in_dim = state.shape
    action_dim = w3.shape[1]

    p = _PACK
    padded_batch = _round_up(batch, p * 8)
    if padded_batch != batch:
        state = jnp.pad(state, ((0, padded_batch - batch), (0, 0)))
    mp = padded_batch // p

    tm = min(_TM, mp)
    padded_mp = _round_up(mp, tm)
    if padded_mp != mp:
        state = jnp.pad(state, ((0, (padded_mp - mp) * p), (0, 0)))
        mp = padded_mp

    xp = state.reshape(mp, p * in_dim)

    eye = jnp.eye(p, dtype=jnp.float32)
    w1b = jnp.kron(eye, w1)
    w2b = jnp.kron(eye, w2)
    w3b = jnp.kron(eye, w3)
    b1b = jnp.tile(b1, (1, p))
    b2b = jnp.tile(b2, (1, p))
    b3b = jnp.tile(b3, (1, p))

    grid = (mp // tm,)

    def resident(shape):
        return pl.BlockSpec(shape, lambda i, _s=shape: (0,) * len(_s))

    out = pl.pallas_call(
        _mlp_kernel,
        out_shape=jax.ShapeDtypeStruct((mp, p * action_dim), jnp.float32),
        grid=grid,
        in_specs=[
            pl.BlockSpec((tm, p * in_dim), lambda i: (i, 0)),
            resident(w1b.shape), resident(b1b.shape),
            resident(w2b.shape), resident(b2b.shape),
            resident(w3b.shape), resident(b3b.shape),
        ],
        out_specs=pl.BlockSpec((tm, p * action_dim), lambda i: (i, 0)),
        compiler_params=pltpu.CompilerParams(
            dimension_semantics=("parallel",),
            vmem_limit_bytes=64 * 1024 * 1024,
        ),
    )(xp, w1b, b1b, w2b, b2b, w3b, b3b)

    return out.reshape(mp * p, action_dim)[:batch]


def kernel(state, w1, b1, w2, b2, w3, b3):
    return _actor_forward(state, w1, b1, w2, b2, w3, b3)
